# baseline (speedup 1.0000x reference)
import functools

import jax
import jax.numpy as jnp
from jax import lax
from jax.experimental import pallas as pl
from jax.experimental.pallas import tpu as pltpu
from jax.experimental.pallas import tpu_sc as plsc

_NW = 32
_LANES = 16
_ROWLANES = 512
_CHR = 16


def _sc_partials_body(x_hbm, p_hbm, n_hbm, out_hbm, xb, pb, nb, accv, sem0, sem1):
    nc = 2
    w = lax.axis_index("s") * nc + lax.axis_index("c")
    rows_total = x_hbm.shape[0]
    rows_per_w = rows_total // _NW
    n_chunks = rows_per_w // _CHR
    base = w * rows_per_w
    b = w // 2
    h = w % 2

    sems = (sem0, sem1)
    bufs = ((xb, x_hbm), (pb, p_hbm), (nb, n_hbm))

    def start(ci):
        slot = ci % 2
        off = base + ci * _CHR
        return [
            pltpu.async_copy(hbm.at[pl.ds(off, _CHR)], buf.at[slot], sems[slot])
            for (buf, hbm) in bufs
        ]

    inflight = {0: start(0)}

    n_acc = 4
    accs = [jnp.zeros((_LANES,), jnp.float32) for _ in range(2 * n_acc)]

    for ci in range(n_chunks):
        if ci + 1 < n_chunks:
            inflight[ci + 1] = start(ci + 1)
        for hdl in inflight.pop(ci):
            hdl.wait()
        slot = ci % 2

        def body(i, carry):
            acc = list(carry)
            for u in range(_ROWLANES // _LANES):
                xv = xb[slot, i, pl.ds(u * _LANES, _LANES)]
                pv = pb[slot, i, pl.ds(u * _LANES, _LANES)]
                nv = nb[slot, i, pl.ds(u * _LANES, _LANES)]
                mf = jnp.where(jnp.minimum(pv, nv) > 0.5, 1.0, 0.0)
                probs = 1.0 / (1.0 + jnp.exp(-xv))
                k = u % n_acc
                acc[k] = acc[k] + probs * mf
                acc[n_acc + k] = acc[n_acc + k] + mf
            return tuple(acc)

        accs = list(lax.fori_loop(0, _CHR, body, tuple(accs)))

    acc_s = accs[0] + accs[1] + accs[2] + accs[3]
    acc_c = accs[4] + accs[5] + accs[6] + accs[7]
    accv[0, :] = acc_s
    accv[1, :] = acc_c
    pltpu.sync_copy(accv, out_hbm.at[h, b])


def _make_sc_partials(batches):
    return functools.partial(
        pl.kernel,
        out_type=jax.ShapeDtypeStruct((2, batches, 2, _LANES), jnp.float32),
        mesh=plsc.VectorSubcoreMesh(core_axis_name="c", subcore_axis_name="s"),
        scratch_types=[
            pltpu.VMEM((2, _CHR, _ROWLANES), jnp.float32),
            pltpu.VMEM((2, _CHR, _ROWLANES), jnp.float32),
            pltpu.VMEM((2, _CHR, _ROWLANES), jnp.float32),
            pltpu.VMEM((2, _LANES), jnp.float32),
            pltpu.SemaphoreType.DMA,
            pltpu.SemaphoreType.DMA,
        ],
    )(_sc_partials_body)


def _combine_body(inv_ref, part_ref, out_ref):
    part = part_ref[...]
    a = part[0] + part[1]
    red = jnp.sum(a, axis=2)
    p = red[:, 0:1] / red[:, 1:2]
    inv = inv_ref[...]
    terms = -inv * jnp.log(p) - (1.0 - inv) * jnp.log(1.0 - p)
    out_ref[...] = jnp.sum(terms).reshape(1, 1)


def kernel(cancer_logits, prostate_mask, needle_mask, involvement, grade_group):
    B, _, H, W = cancer_logits.shape
    x = cancer_logits.reshape(B * H * W // _ROWLANES, _ROWLANES)
    pm = prostate_mask.reshape(B * H * W // _ROWLANES, _ROWLANES)
    nm = needle_mask.reshape(B * H * W // _ROWLANES, _ROWLANES)

    part = _make_sc_partials(B)(x, pm, nm)

    out = pl.pallas_call(
        _combine_body,
        in_specs=[
            pl.BlockSpec(memory_space=pltpu.VMEM),
            pl.BlockSpec(memory_space=pltpu.VMEM),
        ],
        out_specs=pl.BlockSpec(memory_space=pltpu.VMEM),
        out_shape=jax.ShapeDtypeStruct((1, 1), jnp.float32),
    )(involvement.reshape(B, 1), part)
    return out[0, 0]


# speedup vs baseline: 1.7965x; 1.7965x over previous
import functools

import jax
import jax.numpy as jnp
from jax import lax
from jax.experimental import pallas as pl
from jax.experimental.pallas import tpu as pltpu
from jax.experimental.pallas import tpu_sc as plsc

_NW = 32
_LANES = 16
_W = 512
_CHR = 32
_BUFR = _CHR * (_W // 128)


def _sc_partials_body(x_hbm, p_hbm, n_hbm, out_hbm, xb, pb, nb, accv, sem0, sem1):
    nc = 2
    w = lax.axis_index("s") * nc + lax.axis_index("c")
    rows_total = x_hbm.shape[0]
    rows_per_w = rows_total // _NW
    n_chunks = rows_per_w // _CHR
    base = w * rows_per_w
    b = w // 2
    h = w % 2

    sems = (sem0, sem1)
    bufs = ((xb, x_hbm), (pb, p_hbm), (nb, n_hbm))

    def start(ci):
        slot = ci % 2
        off = base + ci * _CHR
        return [
            pltpu.async_copy(
                hbm.at[pl.ds(off, _CHR), pl.ds(t * 128, 128)],
                buf.at[slot, pl.ds(t * _CHR, _CHR)],
                sems[slot],
            )
            for (buf, hbm) in bufs
            for t in range(_W // 128)
        ]

    inflight = {0: start(0)}

    acc_s = jnp.zeros((_LANES,), jnp.float32)
    acc_c = jnp.zeros((_LANES,), jnp.float32)

    for ci in range(n_chunks):
        if ci + 1 < n_chunks:
            inflight[ci + 1] = start(ci + 1)
        for hdl in inflight.pop(ci):
            hdl.wait()
        slot = ci % 2

        def body(i, carry):
            a_s, a_c = carry
            for u in range(128 // _LANES):
                xv = xb[slot, i, pl.ds(u * _LANES, _LANES)]
                pv = pb[slot, i, pl.ds(u * _LANES, _LANES)]
                nv = nb[slot, i, pl.ds(u * _LANES, _LANES)]
                mf = jnp.where(jnp.minimum(pv, nv) > 0.5, 1.0, 0.0)
                probs = 1.0 / (1.0 + jnp.exp(-xv))
                a_s = a_s + probs * mf
                a_c = a_c + mf
            return a_s, a_c

        acc_s, acc_c = lax.fori_loop(0, _BUFR, body, (acc_s, acc_c))

    accv[0, :] = acc_s
    accv[1, :] = acc_c
    pltpu.sync_copy(accv, out_hbm.at[h, b])


def _make_sc_partials(batches):
    return functools.partial(
        pl.kernel,
        out_type=jax.ShapeDtypeStruct((2, batches, 2, _LANES), jnp.float32),
        mesh=plsc.VectorSubcoreMesh(core_axis_name="c", subcore_axis_name="s"),
        scratch_types=[
            pltpu.VMEM((2, _BUFR, 128), jnp.float32),
            pltpu.VMEM((2, _BUFR, 128), jnp.float32),
            pltpu.VMEM((2, _BUFR, 128), jnp.float32),
            pltpu.VMEM((2, _LANES), jnp.float32),
            pltpu.SemaphoreType.DMA,
            pltpu.SemaphoreType.DMA,
        ],
    )(_sc_partials_body)


def _combine_body(inv_ref, part_ref, out_ref):
    part = part_ref[...]
    a = part[0] + part[1]
    red = jnp.sum(a, axis=2)
    p = red[:, 0:1] / red[:, 1:2]
    inv = inv_ref[...]
    terms = -inv * jnp.log(p) - (1.0 - inv) * jnp.log(1.0 - p)
    out_ref[...] = jnp.sum(terms).reshape(1, 1)


def kernel(cancer_logits, prostate_mask, needle_mask, involvement, grade_group):
    B, _, H, W = cancer_logits.shape
    x = cancer_logits.reshape(B * H, W)
    pm = prostate_mask.reshape(B * H, W)
    nm = needle_mask.reshape(B * H, W)

    part = _make_sc_partials(B)(x, pm, nm)

    out = pl.pallas_call(
        _combine_body,
        in_specs=[
            pl.BlockSpec(memory_space=pltpu.VMEM),
            pl.BlockSpec(memory_space=pltpu.VMEM),
        ],
        out_specs=pl.BlockSpec(memory_space=pltpu.VMEM),
        out_shape=jax.ShapeDtypeStruct((1, 1), jnp.float32),
    )(involvement.reshape(B, 1), part)
    return out[0, 0]


# speedup vs baseline: 2.3113x; 1.2865x over previous
import functools

import jax
import jax.numpy as jnp
from jax import lax
from jax.experimental import pallas as pl
from jax.experimental.pallas import tpu as pltpu
from jax.experimental.pallas import tpu_sc as plsc

_NW = 32
_LANES = 16
_W = 512
_ROWS_PER_B = 512
_NB_TC = 4


def _sc_partials_body(nsc, start_row, x_hbm, p_hbm, n_hbm, out_hbm,
                      xb, pb, nb, accv, sem0, sem1):
    nc = 2
    w = lax.axis_index("s") * nc + lax.axis_index("c")
    rows_per_w = nsc * _ROWS_PER_B // _NW
    chr_ = min(32, rows_per_w)
    n_chunks = rows_per_w // chr_
    base = start_row + w * rows_per_w
    workers_per_b = _NW // nsc
    b = w // workers_per_b
    h = w % workers_per_b

    sems = (sem0, sem1)
    bufs = ((xb, x_hbm), (pb, p_hbm), (nb, n_hbm))

    def start(ci):
        slot = ci % 2
        off = base + ci * chr_
        return [
            pltpu.async_copy(
                hbm.at[pl.ds(off, chr_), pl.ds(t * 128, 128)],
                buf.at[slot, pl.ds(t * chr_, chr_)],
                sems[slot],
            )
            for (buf, hbm) in bufs
            for t in range(_W // 128)
        ]

    inflight = {0: start(0)}

    acc_s = jnp.zeros((_LANES,), jnp.float32)
    acc_c = jnp.zeros((_LANES,), jnp.float32)

    for ci in range(n_chunks):
        if ci + 1 < n_chunks:
            inflight[ci + 1] = start(ci + 1)
        for hdl in inflight.pop(ci):
            hdl.wait()
        slot = ci % 2

        def body(i, carry):
            a_s, a_c = carry
            for u in range(128 // _LANES):
                xv = xb[slot, i, pl.ds(u * _LANES, _LANES)]
                pv = pb[slot, i, pl.ds(u * _LANES, _LANES)]
                nv = nb[slot, i, pl.ds(u * _LANES, _LANES)]
                mf = jnp.where(jnp.minimum(pv, nv) > 0.5, 1.0, 0.0)
                probs = 1.0 / (1.0 + jnp.exp(-xv))
                a_s = a_s + probs * mf
                a_c = a_c + mf
            return a_s, a_c

        acc_s, acc_c = lax.fori_loop(0, chr_ * (_W // 128), body, (acc_s, acc_c))

    accv[0, :] = acc_s
    accv[1, :] = acc_c
    pltpu.sync_copy(accv, out_hbm.at[h, b])


def _make_sc_partials(nsc, start_row, buf_rows):
    return functools.partial(
        pl.kernel,
        out_type=jax.ShapeDtypeStruct((_NW // nsc, nsc, 2, _LANES), jnp.float32),
        mesh=plsc.VectorSubcoreMesh(core_axis_name="c", subcore_axis_name="s"),
        scratch_types=[
            pltpu.VMEM((2, buf_rows, 128), jnp.float32),
            pltpu.VMEM((2, buf_rows, 128), jnp.float32),
            pltpu.VMEM((2, buf_rows, 128), jnp.float32),
            pltpu.VMEM((2, _LANES), jnp.float32),
            pltpu.SemaphoreType.DMA,
            pltpu.SemaphoreType.DMA,
        ],
    )(functools.partial(_sc_partials_body, nsc, start_row))


def _tc_body(inv_ref, x_ref, p_ref, n_ref, out_ref):
    g = pl.program_id(0)

    m = (p_ref[...] > 0.5) & (n_ref[...] > 0.5)
    mf = m.astype(jnp.float32)
    probs = jax.nn.sigmoid(x_ref[...])
    mp = probs * mf

    total = jnp.float32(0.0)
    for j in range(_NB_TC):
        ps = jnp.sum(mp[j])
        pc = jnp.sum(mf[j])
        p = ps / pc
        inv = inv_ref[g * _NB_TC + j]
        total += -inv * jnp.log(p) - (1.0 - inv) * jnp.log(1.0 - p)

    @pl.when(g == 0)
    def _():
        out_ref[...] = jnp.zeros_like(out_ref)

    out_ref[...] = out_ref[...] + total


def _combine_body(inv_ref, tc_ref, part_ref, out_ref):
    part = part_ref[...]
    a = jnp.sum(part, axis=0)
    red = jnp.sum(a, axis=2)
    p = red[:, 0:1] / red[:, 1:2]
    inv = inv_ref[...]
    terms = -inv * jnp.log(p) - (1.0 - inv) * jnp.log(1.0 - p)
    out_ref[...] = tc_ref[...] + jnp.sum(terms).reshape(1, 1)


def kernel(cancer_logits, prostate_mask, needle_mask, involvement, grade_group):
    B, _, H, W = cancer_logits.shape
    nsc = 4
    ntc = B - nsc
    x3 = cancer_logits.reshape(B, H, W)
    pm3 = prostate_mask.reshape(B, H, W)
    nm3 = needle_mask.reshape(B, H, W)
    x2 = cancer_logits.reshape(B * H, W)
    pm2 = prostate_mask.reshape(B * H, W)
    nm2 = needle_mask.reshape(B * H, W)

    rows_per_w = nsc * _ROWS_PER_B // _NW
    buf_rows = min(32, rows_per_w) * (_W // 128)
    part = _make_sc_partials(nsc, ntc * _ROWS_PER_B, buf_rows)(x2, pm2, nm2)

    img_spec = pl.BlockSpec((_NB_TC, H, W), lambda g: (g, 0, 0))
    tc_out = pl.pallas_call(
        _tc_body,
        grid=(ntc // _NB_TC,),
        in_specs=[
            pl.BlockSpec(memory_space=pltpu.SMEM),
            img_spec,
            img_spec,
            img_spec,
        ],
        out_specs=pl.BlockSpec((1, 1), lambda g: (0, 0)),
        out_shape=jax.ShapeDtypeStruct((1, 1), jnp.float32),
    )(involvement, x3, pm3, nm3)

    out = pl.pallas_call(
        _combine_body,
        in_specs=[
            pl.BlockSpec(memory_space=pltpu.VMEM),
            pl.BlockSpec(memory_space=pltpu.VMEM),
            pl.BlockSpec(memory_space=pltpu.VMEM),
        ],
        out_specs=pl.BlockSpec(memory_space=pltpu.VMEM),
        out_shape=jax.ShapeDtypeStruct((1, 1), jnp.float32),
    )(involvement[ntc:].reshape(nsc, 1), tc_out, part)
    return out[0, 0]


# speedup vs baseline: 4.8985x; 2.1193x over previous
import functools

import jax
import jax.numpy as jnp
from jax import lax
from jax.experimental import pallas as pl
from jax.experimental.pallas import tpu as pltpu

_NSLOT = 3
_CH_ROWS = 512


def _mil_body(inv_ref, x_hbm, p_hbm, n_hbm, out_ref, xb, pb, nb, s0, s1, s2):
    n_chunks = x_hbm.shape[0] // _CH_ROWS
    sems = (s0, s1, s2)
    bufs = ((xb, x_hbm), (pb, p_hbm), (nb, n_hbm))

    def copies(ci, k):
        return [
            pltpu.make_async_copy(hbm.at[pl.ds(ci * _CH_ROWS, _CH_ROWS)], buf.at[k], sems[k])
            for (buf, hbm) in bufs
        ]

    for k in range(_NSLOT):
        for c in copies(k, k):
            c.start()

    def body(ci, total):
        slot = lax.rem(ci, _NSLOT)
        for k in range(_NSLOT):
            @pl.when(slot == k)
            def _():
                for c in copies(ci, k):
                    c.wait()

        xv = xb[slot]
        m = (pb[slot] > 0.5) & (nb[slot] > 0.5)
        mf = m.astype(jnp.float32)
        probs = jax.nn.sigmoid(xv)
        ps = jnp.sum(probs * mf)
        pc = jnp.sum(mf)

        for k in range(_NSLOT):
            @pl.when(jnp.logical_and(slot == k, ci + _NSLOT < n_chunks))
            def _():
                for c in copies(ci + _NSLOT, k):
                    c.start()

        p = ps / pc
        inv = inv_ref[ci]
        return total + (-inv * jnp.log(p) - (1.0 - inv) * jnp.log(1.0 - p))

    total = lax.fori_loop(0, n_chunks, body, jnp.float32(0.0))
    out_ref[...] = total.reshape(1, 1)


def kernel(cancer_logits, prostate_mask, needle_mask, involvement, grade_group):
    B, _, H, W = cancer_logits.shape
    x = cancer_logits.reshape(B * H, W)
    pm = prostate_mask.reshape(B * H, W)
    nm = needle_mask.reshape(B * H, W)

    out = pl.pallas_call(
        _mil_body,
        in_specs=[
            pl.BlockSpec(memory_space=pltpu.SMEM),
            pl.BlockSpec(memory_space=pl.ANY),
            pl.BlockSpec(memory_space=pl.ANY),
            pl.BlockSpec(memory_space=pl.ANY),
        ],
        out_specs=pl.BlockSpec(memory_space=pltpu.VMEM),
        out_shape=jax.ShapeDtypeStruct((1, 1), jnp.float32),
        scratch_shapes=[
            pltpu.VMEM((_NSLOT, _CH_ROWS, W), jnp.float32),
            pltpu.VMEM((_NSLOT, _CH_ROWS, W), jnp.float32),
            pltpu.VMEM((_NSLOT, _CH_ROWS, W), jnp.float32),
            pltpu.SemaphoreType.DMA,
            pltpu.SemaphoreType.DMA,
            pltpu.SemaphoreType.DMA,
        ],
    )(involvement, x, pm, nm)
    return out[0, 0]
